# baseline (speedup 1.0000x reference)
import functools

import jax
import jax.numpy as jnp
from jax import lax
from jax.experimental import pallas as pl
from jax.experimental.pallas import tpu as pltpu
from jax.experimental.pallas import tpu_sc as plsc

_NC = 2
_NS = 16
_NW = _NC * _NS
_LANES = 16

_D = 96
_ROWS = 64
_CHK = 512
_GRP = 128


def _build_table(W_aromatic, W_conjugated, W_stereo):
    r = jnp.arange(_ROWS - 1)
    tab = jnp.concatenate(
        [W_aromatic[r // 21], W_conjugated[(r // 7) % 3], W_stereo[r % 7]],
        axis=1,
    )
    return jnp.concatenate([tab, jnp.zeros((1, _D), jnp.float32)], axis=0)


@functools.partial(jax.jit, static_argnames=("E",))
def _sc_lookup(idx_a, idx_c, idx_s, mask_i32, table, *, E):
    per = E // _NW
    n_chunks = -(-per // _CHK)
    mesh = plsc.VectorSubcoreMesh(core_axis_name="c", subcore_axis_name="s")

    @functools.partial(
        pl.kernel,
        out_type=jax.ShapeDtypeStruct((E, _D), jnp.float32),
        mesh=mesh,
        scratch_types=[
            pltpu.VMEM((_CHK,), jnp.int32),
            pltpu.VMEM((_CHK,), jnp.int32),
            pltpu.VMEM((_CHK,), jnp.int32),
            pltpu.VMEM((_CHK,), jnp.int32),
            pltpu.VMEM((_CHK,), jnp.int32),
            pltpu.VMEM((_CHK, _D), jnp.float32),
            pltpu.SemaphoreType.DMA,
        ],
        compiler_params=pltpu.CompilerParams(use_tc_tiling_on_sc=False),
    )
    def body(a_hbm, c_hbm, s_hbm, m_hbm, tab_hbm, out_hbm,
             a_v, c_v, s_v, m_v, comb_v, rows_v, sem):
        wid = lax.axis_index("s") * _NC + lax.axis_index("c")
        base0 = wid * per

        def chunk_body(i, carry):
            off = jnp.minimum(i * _CHK, per - _CHK)
            base = base0 + off
            pltpu.sync_copy(a_hbm.at[pl.ds(base, _CHK)], a_v)
            pltpu.sync_copy(c_hbm.at[pl.ds(base, _CHK)], c_v)
            pltpu.sync_copy(s_hbm.at[pl.ds(base, _CHK)], s_v)
            pltpu.sync_copy(m_hbm.at[pl.ds(base, _CHK)], m_v)

            def vec_body(j, c2):
                sl = pl.ds(j * _LANES, _LANES)
                comb = a_v[sl] * 21 + c_v[sl] * 7 + s_v[sl]
                comb_v[sl] = jnp.where(m_v[sl] != 0, comb, _ROWS - 1)
                return c2

            lax.fori_loop(0, _CHK // _LANES, vec_body, 0)

            copies = []
            for g in range(_CHK // _GRP):
                copies.append(pltpu.async_copy(
                    tab_hbm.at[comb_v.at[pl.ds(g * _GRP, _GRP)]],
                    rows_v.at[pl.ds(g * _GRP, _GRP), :],
                    sem,
                ))
            for cp in copies:
                cp.wait()
            pltpu.sync_copy(rows_v, out_hbm.at[pl.ds(base, _CHK)])
            return carry

        lax.fori_loop(0, n_chunks, chunk_body, 0)

    return body(idx_a, idx_c, idx_s, mask_i32, table)


def kernel(bond_mask, prop_bond_aromatic, prop_bond_conjugated,
           prop_bond_stereo, W_aromatic, W_conjugated, W_stereo):
    E = bond_mask.shape[0]
    table = _build_table(W_aromatic, W_conjugated, W_stereo)
    return _sc_lookup(
        prop_bond_aromatic.astype(jnp.int32),
        prop_bond_conjugated.astype(jnp.int32),
        prop_bond_stereo.astype(jnp.int32),
        bond_mask.astype(jnp.int32),
        table,
        E=E,
    )


# speedup vs baseline: 3.4426x; 3.4426x over previous
import functools

import jax
import jax.numpy as jnp
from jax import lax
from jax.experimental import pallas as pl
from jax.experimental.pallas import tpu as pltpu
from jax.experimental.pallas import tpu_sc as plsc

_NC = 2
_NS = 16
_NW = _NC * _NS
_L = 16

_D = 96
_ROWS = 64
_CHK = 512


def _build_table(W_aromatic, W_conjugated, W_stereo):
    r = jnp.arange(_ROWS - 1)
    tab = jnp.concatenate(
        [W_aromatic[r // 21], W_conjugated[(r // 7) % 3], W_stereo[r % 7]],
        axis=1,
    )
    tab = jnp.concatenate([tab, jnp.zeros((1, _D), jnp.float32)], axis=0)
    return tab.reshape(-1)


@functools.partial(jax.jit, static_argnames=("E",))
def _sc_lookup(idx_a, idx_c, idx_s, mask_i32, table, *, E):
    per = E // _NW
    n_chunks = -(-per // _CHK)
    n_chunks += n_chunks % 2
    mesh = plsc.VectorSubcoreMesh(core_axis_name="c", subcore_axis_name="s")

    @functools.partial(
        pl.kernel,
        out_type=jax.ShapeDtypeStruct((E * _D,), jnp.float32),
        mesh=mesh,
        scratch_types=[
            pltpu.VMEM((_ROWS * _D,), jnp.float32),
            [pltpu.VMEM((4 * _CHK,), jnp.int32)] * 2,
            [pltpu.VMEM((_CHK * _D,), jnp.float32)] * 2,
            [pltpu.SemaphoreType.DMA] * 2,
            [pltpu.SemaphoreType.DMA] * 2,
            pltpu.SemaphoreType.DMA,
        ],
        compiler_params=pltpu.CompilerParams(needs_layout_passes=False),
    )
    def body(a_hbm, c_hbm, s_hbm, m_hbm, tab_hbm, out_hbm,
             tab_v, idx_v, out_v, isem, osem, tsem):
        wid = lax.axis_index("s") * _NC + lax.axis_index("c")
        base0 = wid * per
        iota = lax.iota(jnp.int32, _L)

        tab_cp = pltpu.async_copy(tab_hbm, tab_v, tsem)

        def chunk_off(j):
            return jnp.minimum(j * _CHK, per - _CHK)

        def issue_idx(j, b):
            base = base0 + chunk_off(j)
            cps = []
            for q, src in enumerate((a_hbm, c_hbm, s_hbm, m_hbm)):
                cps.append(pltpu.async_copy(
                    src.at[pl.ds(base, _CHK)],
                    idx_v[b].at[pl.ds(q * _CHK, _CHK)],
                    isem[b],
                ))
            return cps

        issue_idx(0, 0)
        tab_cp.wait()

        @pl.loop(0, n_chunks, step=2)
        def chunk_pair(i):
            for b in (0, 1):
                j = i + b
                issue_idx(j + 1, 1 - b)
                for _ in range(4):
                    pltpu.make_async_copy(
                        a_hbm.at[pl.ds(0, _CHK)],
                        idx_v[b].at[pl.ds(0, _CHK)],
                        isem[b],
                    ).wait()
                @pl.when(j >= 2)
                def _():
                    pltpu.make_async_copy(
                        out_v[b],
                        out_hbm.at[pl.ds(0, _CHK * _D)],
                        osem[b],
                    ).wait()

                def group(g, carry):
                    sl = lambda q: pl.ds(q * _CHK + g * _L, _L)
                    comb = (idx_v[b][sl(0)] * 21 + idx_v[b][sl(1)] * 7
                            + idx_v[b][sl(2)])
                    comb = jnp.where(idx_v[b][sl(3)] != 0, comb, _ROWS - 1)
                    src0 = comb * _D
                    dst0 = iota * _D + g * (_L * _D)
                    for k in range(_D):
                        v = plsc.load_gather(tab_v, [src0 + k])
                        plsc.store_scatter(out_v[b], [dst0 + k], v)
                    return carry

                lax.fori_loop(0, _CHK // _L, group, 0)

                base = base0 + chunk_off(j)
                pltpu.async_copy(
                    out_v[b],
                    out_hbm.at[pl.ds(base * _D, _CHK * _D)],
                    osem[b],
                )

        for b in (0, 1):
            pltpu.make_async_copy(
                out_v[b], out_hbm.at[pl.ds(0, _CHK * _D)], osem[b],
            ).wait()
        for _ in range(4):
            pltpu.make_async_copy(
                a_hbm.at[pl.ds(0, _CHK)],
                idx_v[0].at[pl.ds(0, _CHK)],
                isem[0],
            ).wait()

    return body(idx_a, idx_c, idx_s, mask_i32, table)


def kernel(bond_mask, prop_bond_aromatic, prop_bond_conjugated,
           prop_bond_stereo, W_aromatic, W_conjugated, W_stereo):
    E = bond_mask.shape[0]
    table = _build_table(W_aromatic, W_conjugated, W_stereo)
    flat = _sc_lookup(
        prop_bond_aromatic.astype(jnp.int32),
        prop_bond_conjugated.astype(jnp.int32),
        prop_bond_stereo.astype(jnp.int32),
        bond_mask.astype(jnp.int32),
        table,
        E=E,
    )
    return flat.reshape(E, _D)


# speedup vs baseline: 16.2085x; 4.7082x over previous
import functools

import jax
import jax.numpy as jnp
from jax import lax
from jax.experimental import pallas as pl
from jax.experimental.pallas import tpu as pltpu
from jax.experimental.pallas import tpu_sc as plsc

_NC = 2
_NS = 16
_NW = _NC * _NS
_L = 16

_D = 96
_ROWS = 64
_CHK = 512


def _build_table(W_aromatic, W_conjugated, W_stereo):
    r = jnp.arange(_ROWS - 1)
    tab = jnp.concatenate(
        [W_aromatic[r // 21], W_conjugated[(r // 7) % 3], W_stereo[r % 7]],
        axis=1,
    )
    tab = jnp.concatenate([tab, jnp.zeros((1, _D), jnp.float32)], axis=0)
    return tab.reshape(-1)


@functools.partial(jax.jit, static_argnames=("E",))
def _sc_lookup(idx_a, idx_c, idx_s, mask_i32, table, *, E):
    per = E // _NW
    n_chunks = -(-per // _CHK)
    n_chunks += n_chunks % 2
    mesh = plsc.VectorSubcoreMesh(core_axis_name="c", subcore_axis_name="s")

    @functools.partial(
        pl.kernel,
        out_type=jax.ShapeDtypeStruct((E * _D,), jnp.float32),
        mesh=mesh,
        scratch_types=[
            pltpu.VMEM((_ROWS * _D,), jnp.float32),
            [pltpu.VMEM((4 * _CHK,), jnp.int32)] * 2,
            [pltpu.VMEM((_CHK * _D,), jnp.float32)] * 2,
            [pltpu.SemaphoreType.DMA] * 2,
            [pltpu.SemaphoreType.DMA] * 2,
            pltpu.SemaphoreType.DMA,
        ],
        compiler_params=pltpu.CompilerParams(
            needs_layout_passes=False, disable_bounds_checks=True),
    )
    def body(a_hbm, c_hbm, s_hbm, m_hbm, tab_hbm, out_hbm,
             tab_v, idx_v, out_v, isem, osem, tsem):
        wid = lax.axis_index("s") * _NC + lax.axis_index("c")
        base0 = wid * per

        tab_cp = pltpu.async_copy(tab_hbm, tab_v, tsem)

        def chunk_off(j):
            return jnp.minimum(j * _CHK, per - _CHK)

        def issue_idx(j, b):
            base = base0 + chunk_off(j)
            for q, src in enumerate((a_hbm, c_hbm, s_hbm, m_hbm)):
                pltpu.async_copy(
                    src.at[pl.ds(base, _CHK)],
                    idx_v[b].at[pl.ds(q * _CHK, _CHK)],
                    isem[b],
                )

        issue_idx(0, 0)
        tab_cp.wait()

        @pl.loop(0, n_chunks, step=2)
        def chunk_pair(i):
            for b in (0, 1):
                j = i + b
                issue_idx(j + 1, 1 - b)
                for _ in range(4):
                    pltpu.make_async_copy(
                        a_hbm.at[pl.ds(0, _CHK)],
                        idx_v[b].at[pl.ds(0, _CHK)],
                        isem[b],
                    ).wait()

                @pl.when(j >= 2)
                def _():
                    pltpu.make_async_copy(
                        out_v[b],
                        out_hbm.at[pl.ds(0, _CHK * _D)],
                        osem[b],
                    ).wait()

                @plsc.parallel_loop(0, _CHK // _L)
                def group_loop(g):
                    sl = lambda q: pl.ds(q * _CHK + g * _L, _L)
                    comb = (idx_v[b][sl(0)] * 21 + idx_v[b][sl(1)] * 7
                            + idx_v[b][sl(2)])
                    comb = jnp.where(idx_v[b][sl(3)] != 0, comb,
                                     _ROWS - 1) * _D
                    for r in range(_L):
                        src = comb[r]
                        dst = (g * _L + r) * _D
                        for k in range(_D // _L):
                            out_v[b][pl.ds(dst + k * _L, _L)] = (
                                tab_v[pl.ds(src + k * _L, _L)])

                base = base0 + chunk_off(j)
                pltpu.async_copy(
                    out_v[b],
                    out_hbm.at[pl.ds(base * _D, _CHK * _D)],
                    osem[b],
                )

        for b in (0, 1):
            pltpu.make_async_copy(
                out_v[b], out_hbm.at[pl.ds(0, _CHK * _D)], osem[b],
            ).wait()
        for _ in range(4):
            pltpu.make_async_copy(
                a_hbm.at[pl.ds(0, _CHK)],
                idx_v[0].at[pl.ds(0, _CHK)],
                isem[0],
            ).wait()

    return body(idx_a, idx_c, idx_s, mask_i32, table)


def kernel(bond_mask, prop_bond_aromatic, prop_bond_conjugated,
           prop_bond_stereo, W_aromatic, W_conjugated, W_stereo):
    E = bond_mask.shape[0]
    table = _build_table(W_aromatic, W_conjugated, W_stereo)
    flat = _sc_lookup(
        prop_bond_aromatic.astype(jnp.int32),
        prop_bond_conjugated.astype(jnp.int32),
        prop_bond_stereo.astype(jnp.int32),
        bond_mask.astype(jnp.int32),
        table,
        E=E,
    )
    return flat.reshape(E, _D)


# speedup vs baseline: 25.6876x; 1.5848x over previous
import functools

import jax
import jax.numpy as jnp
from jax import lax
from jax.experimental import pallas as pl
from jax.experimental.pallas import tpu as pltpu
from jax.experimental.pallas import tpu_sc as plsc

_NC = 2
_NS = 16
_NW = _NC * _NS
_L = 16

_D = 96
_DP = 128
_ROWS = 64
_CHK = 448


def _build_table(W_aromatic, W_conjugated, W_stereo):
    r = jnp.arange(_ROWS - 1)
    tab = jnp.concatenate(
        [W_aromatic[r // 21], W_conjugated[(r // 7) % 3], W_stereo[r % 7]],
        axis=1,
    )
    tab = jnp.concatenate([tab, jnp.zeros((1, _D), jnp.float32)], axis=0)
    return tab.reshape(-1)


@functools.partial(jax.jit, static_argnames=("E",))
def _sc_lookup(idx_a, idx_c, idx_s, mask_i32, table, *, E):
    per = E // _NW
    n_chunks = -(-per // _CHK)
    n_chunks += n_chunks % 2
    mesh = plsc.VectorSubcoreMesh(core_axis_name="c", subcore_axis_name="s")

    @functools.partial(
        pl.kernel,
        out_type=jax.ShapeDtypeStruct((E, _D), jnp.float32),
        mesh=mesh,
        scratch_types=[
            pltpu.VMEM((_ROWS * _D,), jnp.float32),
            [pltpu.VMEM((4 * _CHK,), jnp.int32)] * 2,
            [pltpu.VMEM((_CHK, _D), jnp.float32)] * 2,
            [pltpu.SemaphoreType.DMA] * 2,
            [pltpu.SemaphoreType.DMA] * 2,
            pltpu.SemaphoreType.DMA,
        ],
        compiler_params=pltpu.CompilerParams(
            needs_layout_passes=False, disable_bounds_checks=True),
    )
    def body(a_hbm, c_hbm, s_hbm, m_hbm, tab_hbm, out_hbm,
             tab_v, idx_v, out_v, isem, osem, tsem):
        wid = lax.axis_index("s") * _NC + lax.axis_index("c")
        base0 = wid * per

        tab_cp = pltpu.async_copy(tab_hbm, tab_v, tsem)

        def chunk_off(j):
            return jnp.minimum(j * _CHK, per - _CHK)

        def issue_idx(j, b):
            base = base0 + chunk_off(j)
            for q, src in enumerate((a_hbm, c_hbm, s_hbm, m_hbm)):
                pltpu.async_copy(
                    src.at[pl.ds(base, _CHK)],
                    idx_v[b].at[pl.ds(q * _CHK, _CHK)],
                    isem[b],
                )

        issue_idx(0, 0)
        tab_cp.wait()

        @pl.loop(0, n_chunks, step=2)
        def chunk_pair(i):
            for b in (0, 1):
                j = i + b
                issue_idx(j + 1, 1 - b)
                for _ in range(4):
                    pltpu.make_async_copy(
                        a_hbm.at[pl.ds(0, _CHK)],
                        idx_v[b].at[pl.ds(0, _CHK)],
                        isem[b],
                    ).wait()

                @pl.when(j >= 2)
                def _():
                    pltpu.make_async_copy(
                        out_v[b],
                        out_hbm.at[pl.ds(0, _CHK)],
                        osem[b],
                    ).wait()

                @plsc.parallel_loop(0, _CHK // _L)
                def group_loop(g):
                    sl = lambda q: pl.ds(q * _CHK + g * _L, _L)
                    comb = (idx_v[b][sl(0)] * 21 + idx_v[b][sl(1)] * 7
                            + idx_v[b][sl(2)])
                    comb = jnp.where(idx_v[b][sl(3)] != 0, comb,
                                     _ROWS - 1) * _D
                    for r in range(_L):
                        src = comb[r]
                        row = g * _L + r
                        for k in range(_D // _L):
                            out_v[b][row, pl.ds(k * _L, _L)] = (
                                tab_v[pl.ds(src + k * _L, _L)])

                base = base0 + chunk_off(j)
                pltpu.async_copy(
                    out_v[b],
                    out_hbm.at[pl.ds(base, _CHK)],
                    osem[b],
                )

        for b in (0, 1):
            pltpu.make_async_copy(
                out_v[b], out_hbm.at[pl.ds(0, _CHK)], osem[b],
            ).wait()
        for _ in range(4):
            pltpu.make_async_copy(
                a_hbm.at[pl.ds(0, _CHK)],
                idx_v[0].at[pl.ds(0, _CHK)],
                isem[0],
            ).wait()

    return body(idx_a, idx_c, idx_s, mask_i32, table)


def kernel(bond_mask, prop_bond_aromatic, prop_bond_conjugated,
           prop_bond_stereo, W_aromatic, W_conjugated, W_stereo):
    E = bond_mask.shape[0]
    table = _build_table(W_aromatic, W_conjugated, W_stereo)
    return _sc_lookup(
        prop_bond_aromatic.astype(jnp.int32),
        prop_bond_conjugated.astype(jnp.int32),
        prop_bond_stereo.astype(jnp.int32),
        bond_mask.astype(jnp.int32),
        table,
        E=E,
    )


# speedup vs baseline: 46.4806x; 1.8095x over previous
import functools

import jax
import jax.numpy as jnp
from jax import lax
from jax.experimental import pallas as pl
from jax.experimental.pallas import tpu as pltpu
from jax.experimental.pallas import tpu_sc as plsc

_NC = 2
_NS = 16
_NW = _NC * _NS
_L = 16

_D = 96
_ROWS = 64
_CHK = 512


def _build_table_t(W_aromatic, W_conjugated, W_stereo):
    r = jnp.arange(_ROWS - 1)
    tab = jnp.concatenate(
        [W_aromatic[r // 21], W_conjugated[(r // 7) % 3], W_stereo[r % 7]],
        axis=1,
    )
    tab = jnp.concatenate([tab, jnp.zeros((1, _D), jnp.float32)], axis=0)
    return tab.T.reshape(-1)


@functools.partial(jax.jit, static_argnames=("E",))
def _sc_lookup(idx_a, idx_c, idx_s, mask_i32, table_t, *, E):
    n_total = -(-E // _CHK)
    last = n_total - 1
    n_per_w = -(-n_total // _NW)
    n_per_w += n_per_w % 2
    mesh = plsc.VectorSubcoreMesh(core_axis_name="c", subcore_axis_name="s")

    @functools.partial(
        pl.kernel,
        out_type=jax.ShapeDtypeStruct((_D, E), jnp.float32),
        mesh=mesh,
        scratch_types=[
            pltpu.VMEM((_D * _ROWS,), jnp.float32),
            [pltpu.VMEM((4 * _CHK,), jnp.int32)] * 2,
            [pltpu.VMEM((_D, _CHK), jnp.float32)] * 2,
            [pltpu.SemaphoreType.DMA] * 2,
            [pltpu.SemaphoreType.DMA] * 2,
            pltpu.SemaphoreType.DMA,
        ],
        compiler_params=pltpu.CompilerParams(
            needs_layout_passes=False, disable_bounds_checks=True),
    )
    def body(a_hbm, c_hbm, s_hbm, m_hbm, tab_hbm, out_hbm,
             tab_v, idx_v, out_v, isem, osem, tsem):
        wid = lax.axis_index("s") * _NC + lax.axis_index("c")

        tab_cp = pltpu.async_copy(tab_hbm, tab_v, tsem)

        def chunk_base(i):
            cid = jnp.minimum(wid + i * _NW, last)
            return jnp.minimum(cid * _CHK, E - _CHK)

        def issue_idx(i, b):
            base = chunk_base(i)
            for q, src in enumerate((a_hbm, c_hbm, s_hbm, m_hbm)):
                pltpu.async_copy(
                    src.at[pl.ds(base, _CHK)],
                    idx_v[b].at[pl.ds(q * _CHK, _CHK)],
                    isem[b],
                )

        issue_idx(0, 0)
        tab_cp.wait()

        @pl.loop(0, n_per_w, step=2)
        def chunk_pair(i):
            for b in (0, 1):
                j = i + b
                issue_idx(j + 1, 1 - b)
                for _ in range(4):
                    pltpu.make_async_copy(
                        a_hbm.at[pl.ds(0, _CHK)],
                        idx_v[b].at[pl.ds(0, _CHK)],
                        isem[b],
                    ).wait()

                @pl.when(j >= 2)
                def _():
                    pltpu.make_async_copy(
                        out_v[b],
                        out_hbm.at[:, pl.ds(0, _CHK)],
                        osem[b],
                    ).wait()

                @plsc.parallel_loop(0, _CHK // _L)
                def group_loop(g):
                    sl = lambda q: pl.ds(q * _CHK + g * _L, _L)
                    comb = (idx_v[b][sl(0)] * 21 + idx_v[b][sl(1)] * 7
                            + idx_v[b][sl(2)])
                    comb = jnp.where(idx_v[b][sl(3)] != 0, comb, _ROWS - 1)
                    for c in range(_D):
                        out_v[b][c, pl.ds(g * _L, _L)] = plsc.load_gather(
                            tab_v, [comb + c * _ROWS])

                base = chunk_base(j)
                pltpu.async_copy(
                    out_v[b],
                    out_hbm.at[:, pl.ds(base, _CHK)],
                    osem[b],
                )

        for b in (0, 1):
            pltpu.make_async_copy(
                out_v[b], out_hbm.at[:, pl.ds(0, _CHK)], osem[b],
            ).wait()
        for _ in range(4):
            pltpu.make_async_copy(
                a_hbm.at[pl.ds(0, _CHK)],
                idx_v[0].at[pl.ds(0, _CHK)],
                isem[0],
            ).wait()

    return body(idx_a, idx_c, idx_s, mask_i32, table_t)


def kernel(bond_mask, prop_bond_aromatic, prop_bond_conjugated,
           prop_bond_stereo, W_aromatic, W_conjugated, W_stereo):
    E = bond_mask.shape[0]
    table_t = _build_table_t(W_aromatic, W_conjugated, W_stereo)
    out_t = _sc_lookup(
        prop_bond_aromatic.astype(jnp.int32),
        prop_bond_conjugated.astype(jnp.int32),
        prop_bond_stereo.astype(jnp.int32),
        bond_mask.astype(jnp.int32),
        table_t,
        E=E,
    )
    return out_t.T


# speedup vs baseline: 54.2342x; 1.1668x over previous
import functools

import jax
import jax.numpy as jnp
from jax import lax
from jax.experimental import pallas as pl
from jax.experimental.pallas import tpu as pltpu
from jax.experimental.pallas import tpu_sc as plsc

_NC = 2
_NS = 16
_NW = _NC * _NS
_L = 16

_D = 96
_ROWS = 64
_CHK = 512


def _build_table_t(W_aromatic, W_conjugated, W_stereo):
    r = jnp.arange(_ROWS - 1)
    tab = jnp.concatenate(
        [W_aromatic[r // 21], W_conjugated[(r // 7) % 3], W_stereo[r % 7]],
        axis=1,
    )
    tab = jnp.concatenate([tab, jnp.zeros((1, _D), jnp.float32)], axis=0)
    return tab.T.reshape(-1)


@functools.partial(jax.jit, static_argnames=("E",))
def _sc_lookup(idx_a, idx_c, idx_s, mask_i32, table_t, *, E):
    n_total = -(-E // _CHK)
    last = n_total - 1
    n_per_w = -(-n_total // _NW)
    n_per_w += n_per_w % 2
    mesh = plsc.VectorSubcoreMesh(core_axis_name="c", subcore_axis_name="s")

    @functools.partial(
        pl.kernel,
        out_type=jax.ShapeDtypeStruct((_D, E), jnp.float32),
        mesh=mesh,
        scratch_types=[
            pltpu.VMEM((_D * _ROWS,), jnp.float32),
            [pltpu.VMEM((4 * _CHK,), jnp.int32)] * 2,
            [pltpu.VMEM((_D, _CHK), jnp.float32)] * 2,
            [pltpu.SemaphoreType.DMA] * 2,
            [pltpu.SemaphoreType.DMA] * 2,
            pltpu.SemaphoreType.DMA,
        ],
        compiler_params=pltpu.CompilerParams(
            needs_layout_passes=False, disable_bounds_checks=True),
    )
    def body(a_hbm, c_hbm, s_hbm, m_hbm, tab_hbm, out_hbm,
             tab_v, idx_v, out_v, isem, osem, tsem):
        wid = lax.axis_index("s") * _NC + lax.axis_index("c")

        tab_cp = pltpu.async_copy(tab_hbm, tab_v, tsem)

        def chunk_base(i):
            cid = jnp.minimum(wid + i * _NW, last)
            return jnp.minimum(cid * _CHK, E - _CHK)

        def issue_idx(i, b):
            base = chunk_base(i)
            for q, src in enumerate((a_hbm, c_hbm, s_hbm, m_hbm)):
                pltpu.async_copy(
                    src.at[pl.ds(base, _CHK)],
                    idx_v[b].at[pl.ds(q * _CHK, _CHK)],
                    isem[b],
                )

        issue_idx(0, 0)
        tab_cp.wait()

        @pl.loop(0, n_per_w, step=2)
        def chunk_pair(i):
            for b in (0, 1):
                j = i + b
                issue_idx(j + 1, 1 - b)
                for _ in range(4):
                    pltpu.make_async_copy(
                        a_hbm.at[pl.ds(0, _CHK)],
                        idx_v[b].at[pl.ds(0, _CHK)],
                        isem[b],
                    ).wait()

                @pl.when(j >= 2)
                def _():
                    pltpu.make_async_copy(
                        out_v[b],
                        out_hbm.at[:, pl.ds(0, _CHK)],
                        osem[b],
                    ).wait()

                lanes = lax.iota(jnp.int32, _L)

                @plsc.parallel_loop(0, _CHK // _L)
                def group_loop(g):
                    sl = lambda q: pl.ds(q * _CHK + g * _L, _L)
                    comb = (idx_v[b][sl(0)] * 21 + idx_v[b][sl(1)] * 7
                            + idx_v[b][sl(2)])
                    m = idx_v[b][sl(3)]
                    comb = jnp.where(m != 0, comb, lanes)
                    mf = m.astype(jnp.float32)
                    for c in range(_D):
                        out_v[b][c, pl.ds(g * _L, _L)] = mf * plsc.load_gather(
                            tab_v, [comb + c * _ROWS])

                base = chunk_base(j)
                pltpu.async_copy(
                    out_v[b],
                    out_hbm.at[:, pl.ds(base, _CHK)],
                    osem[b],
                )

        for b in (0, 1):
            pltpu.make_async_copy(
                out_v[b], out_hbm.at[:, pl.ds(0, _CHK)], osem[b],
            ).wait()
        for _ in range(4):
            pltpu.make_async_copy(
                a_hbm.at[pl.ds(0, _CHK)],
                idx_v[0].at[pl.ds(0, _CHK)],
                isem[0],
            ).wait()

    return body(idx_a, idx_c, idx_s, mask_i32, table_t)


def kernel(bond_mask, prop_bond_aromatic, prop_bond_conjugated,
           prop_bond_stereo, W_aromatic, W_conjugated, W_stereo):
    E = bond_mask.shape[0]
    table_t = _build_table_t(W_aromatic, W_conjugated, W_stereo)
    out_t = _sc_lookup(
        prop_bond_aromatic.astype(jnp.int32),
        prop_bond_conjugated.astype(jnp.int32),
        prop_bond_stereo.astype(jnp.int32),
        bond_mask.astype(jnp.int32),
        table_t,
        E=E,
    )
    return out_t.T


# speedup vs baseline: 61.4727x; 1.1335x over previous
import functools

import jax
import jax.numpy as jnp
from jax import lax
from jax.experimental import pallas as pl
from jax.experimental.pallas import tpu as pltpu
from jax.experimental.pallas import tpu_sc as plsc

_NC = 2
_NS = 16
_NW = _NC * _NS
_L = 16

_D = 96
_ROWS = 64
_REP = 4
_SEC = _ROWS * _REP + _L
_CHK = 384


def _build_table_t(W_aromatic, W_conjugated, W_stereo):
    r = jnp.arange(_ROWS - 1)
    tab = jnp.concatenate(
        [W_aromatic[r // 21], W_conjugated[(r // 7) % 3], W_stereo[r % 7]],
        axis=1,
    )
    tab = jnp.concatenate([tab, jnp.zeros((1, _D), jnp.float32)], axis=0)
    rep = jnp.repeat(tab.T, _REP, axis=1)
    rep = jnp.concatenate(
        [rep, jnp.zeros((_D, _L), jnp.float32)], axis=1)
    return rep.reshape(-1)


@functools.partial(jax.jit, static_argnames=("E",))
def _sc_lookup(idx_a, idx_c, idx_s, mask_i32, table_t, *, E):
    n_total = -(-E // _CHK)
    last = n_total - 1
    n_per_w = -(-n_total // _NW)
    n_per_w += n_per_w % 2
    mesh = plsc.VectorSubcoreMesh(core_axis_name="c", subcore_axis_name="s")

    @functools.partial(
        pl.kernel,
        out_type=jax.ShapeDtypeStruct((_D, E), jnp.float32),
        mesh=mesh,
        scratch_types=[
            pltpu.VMEM((_D * _SEC,), jnp.float32),
            [pltpu.VMEM((4 * _CHK,), jnp.int32)] * 2,
            [pltpu.VMEM((_D, _CHK), jnp.float32)] * 2,
            [pltpu.SemaphoreType.DMA] * 2,
            [pltpu.SemaphoreType.DMA] * 2,
            pltpu.SemaphoreType.DMA,
        ],
        compiler_params=pltpu.CompilerParams(
            needs_layout_passes=False, disable_bounds_checks=True),
    )
    def body(a_hbm, c_hbm, s_hbm, m_hbm, tab_hbm, out_hbm,
             tab_v, idx_v, out_v, isem, osem, tsem):
        wid = lax.axis_index("s") * _NC + lax.axis_index("c")

        tab_cp = pltpu.async_copy(tab_hbm, tab_v, tsem)

        def chunk_base(i):
            cid = jnp.minimum(wid + i * _NW, last)
            return jnp.minimum(cid * _CHK, E - _CHK)

        def issue_idx(i, b):
            base = chunk_base(i)
            for q, src in enumerate((a_hbm, c_hbm, s_hbm, m_hbm)):
                pltpu.async_copy(
                    src.at[pl.ds(base, _CHK)],
                    idx_v[b].at[pl.ds(q * _CHK, _CHK)],
                    isem[b],
                )

        issue_idx(0, 0)
        tab_cp.wait()

        @pl.loop(0, n_per_w, step=2)
        def chunk_pair(i):
            for b in (0, 1):
                j = i + b
                issue_idx(j + 1, 1 - b)
                for _ in range(4):
                    pltpu.make_async_copy(
                        a_hbm.at[pl.ds(0, _CHK)],
                        idx_v[b].at[pl.ds(0, _CHK)],
                        isem[b],
                    ).wait()

                @pl.when(j >= 2)
                def _():
                    pltpu.make_async_copy(
                        out_v[b],
                        out_hbm.at[:, pl.ds(0, _CHK)],
                        osem[b],
                    ).wait()

                lanes = lax.iota(jnp.int32, _L)

                @plsc.parallel_loop(0, _CHK // _L)
                def group_loop(g):
                    sl = lambda q: pl.ds(q * _CHK + g * _L, _L)
                    comb = (idx_v[b][sl(0)] * 21 + idx_v[b][sl(1)] * 7
                            + idx_v[b][sl(2)]) * _REP + (lanes & (_REP - 1))
                    comb = jnp.where(idx_v[b][sl(3)] != 0, comb,
                                     _ROWS * _REP + lanes)
                    for c in range(_D):
                        out_v[b][c, pl.ds(g * _L, _L)] = plsc.load_gather(
                            tab_v.at[pl.ds(c * _SEC, _SEC)], [comb])

                base = chunk_base(j)
                pltpu.async_copy(
                    out_v[b],
                    out_hbm.at[:, pl.ds(base, _CHK)],
                    osem[b],
                )

        for b in (0, 1):
            pltpu.make_async_copy(
                out_v[b], out_hbm.at[:, pl.ds(0, _CHK)], osem[b],
            ).wait()
        for _ in range(4):
            pltpu.make_async_copy(
                a_hbm.at[pl.ds(0, _CHK)],
                idx_v[0].at[pl.ds(0, _CHK)],
                isem[0],
            ).wait()

    return body(idx_a, idx_c, idx_s, mask_i32, table_t)


def kernel(bond_mask, prop_bond_aromatic, prop_bond_conjugated,
           prop_bond_stereo, W_aromatic, W_conjugated, W_stereo):
    E = bond_mask.shape[0]
    table_t = _build_table_t(W_aromatic, W_conjugated, W_stereo)
    out_t = _sc_lookup(
        prop_bond_aromatic.astype(jnp.int32),
        prop_bond_conjugated.astype(jnp.int32),
        prop_bond_stereo.astype(jnp.int32),
        bond_mask.astype(jnp.int32),
        table_t,
        E=E,
    )
    return out_t.T
